# Initial kernel scaffold; baseline (speedup 1.0000x reference)
import jax
import jax.numpy as jnp
from jax.experimental import pallas as pl


def kernel(input_ids, embed_tokens_weight, audio_tokens_offsets):
    raise NotImplementedError("write your pallas kernel here")


# speedup vs baseline: 3.2786x; 3.2786x over previous
import functools

import jax
import jax.numpy as jnp
from jax import lax
from jax.experimental import pallas as pl
from jax.experimental.pallas import tpu as pltpu
from jax.experimental.pallas import tpu_sc as plsc

B, S = 4, 4096
CPO = 9
H = 1024
HV = H // 16
N = B * S
NC, NS, L = 2, 16, 16
NW = NC * NS
TPW = N // NW
T = 16
NBLK = TPW // T
NSB = 3
SUPER = 4


def _sc_body(ids_hbm, table_hbm, offs_hbm, out_hbm, *scratch):
    idsbuf = scratch[0]
    stage = scratch[1:1 + NSB]
    outbuf = scratch[1 + NSB:3 + NSB]
    offbuf = scratch[3 + NSB]
    sem_s = scratch[4 + NSB:4 + 2 * NSB]
    sem_o = scratch[4 + 2 * NSB:6 + 2 * NSB]

    wid = lax.axis_index("s") * NC + lax.axis_index("c")
    base = wid * TPW

    pltpu.sync_copy(offs_hbm, offbuf)
    lanes = lax.iota(jnp.int32, L)

    def fire_gather(blk, c):
        trow = (blk % SUPER) * T
        cv = jnp.full((L,), c, jnp.int32)
        ids_v = plsc.load_gather(idsbuf, [trow + lanes, cv])
        off_v = offbuf[c, :]
        sb = c % NSB
        return pltpu.async_copy(
            table_hbm.at[ids_v + off_v], stage[sb], sem_s[sb])

    def fold(c, b):
        sref = stage[c % NSB]
        oref = outbuf[b]

        @plsc.parallel_loop(0, HV, step=1, unroll=1)
        def _jloop(j):
            col = pl.ds(j * L, L)
            for t in range(T):
                v = sref[t, col]
                if c == 0:
                    oref[t, col] = v
                else:
                    plsc.addupdate(oref.at[t, col], v)

    def fire_out(blk, b):
        tok = base + blk * T
        return pltpu.async_copy(outbuf[b], out_hbm.at[pl.ds(tok, T)],
                                sem_o[b])

    def wait_out(blk, b):
        tok = base + blk * T
        pltpu.make_async_copy(
            outbuf[b], out_hbm.at[pl.ds(tok, T)], sem_o[b]).wait()

    def do_block(blk, b, first):
        @pl.when(blk % SUPER == 0)
        def _refill_ids():
            tok0 = pl.multiple_of(base + blk * T, SUPER * T)
            pltpu.sync_copy(ids_hbm.at[pl.ds(tok0, SUPER * T)], idsbuf)

        descs = [fire_gather(blk, 0), fire_gather(blk, 1)]

        if not first:
            wait_out(blk - 2, b)

        for c in range(CPO):
            if c + 2 < CPO:
                descs.append(fire_gather(blk, c + 2))
            descs[c].wait()
            fold(c, b)

        return fire_out(blk, b)

    do_block(0, 0, True)
    do_block(1, 1, True)

    def body(ip, carry):
        for b in (0, 1):
            do_block(ip * 2 + b + 2, b, False)
        return carry

    lax.fori_loop(0, (NBLK - 2) // 2, body, 0)
    wait_out(NBLK - 2, 0)
    wait_out(NBLK - 1, 1)


@functools.partial(jax.jit, static_argnums=())
def kernel(input_ids, embed_tokens_weight, audio_tokens_offsets):
    ids2 = input_ids.reshape(N, CPO).astype(jnp.int32)
    offs = jnp.tile(audio_tokens_offsets.astype(jnp.int32)[:, None], (1, L))
    mesh = plsc.VectorSubcoreMesh(core_axis_name="c", subcore_axis_name="s")
    run = pl.kernel(
        _sc_body,
        out_type=jax.ShapeDtypeStruct((N, H), jnp.float32),
        mesh=mesh,
        compiler_params=pltpu.CompilerParams(needs_layout_passes=False),
        scratch_types=(
            [pltpu.VMEM((SUPER * T, CPO), jnp.int32)]
            + [pltpu.VMEM((T, H), jnp.float32)] * NSB
            + [pltpu.VMEM((T, H), jnp.float32)] * 2
            + [pltpu.VMEM((CPO, L), jnp.int32)]
            + [pltpu.SemaphoreType.DMA] * (NSB + 2)
        ),
    )
    out = run(ids2, embed_tokens_weight.astype(jnp.float32), offs)
    return out.reshape(B, S, H)
